# baseline (speedup 1.0000x reference)
import functools

import jax
import jax.numpy as jnp
import numpy as np
from jax import lax
from jax.experimental import pallas as pl
from jax.experimental.pallas import tpu as pltpu
from jax.experimental.pallas import tpu_sc as plsc

N_NODES = 10000
N_EDGES = 320000
X_DIM = 128
EW_DIM = 16
OUT_DIM = 128

NC = 2
NS = 16
NW = NC * NS
E_PER = N_EDGES // NW
CHUNK = 80
N_CHUNKS = E_PER // CHUNK
N_ACC = 10240

E_HALF = N_EDGES // NC
SEG_SUPER = 20000
N_SEG_SUPER = E_HALF // SEG_SUPER

_MESH = plsc.VectorSubcoreMesh(
    core_axis_name="c", subcore_axis_name="s", num_cores=NC, num_subcores=NS)


@functools.partial(
    pl.kernel,
    out_type=jax.ShapeDtypeStruct((NC, EW_DIM, N_ACC), jnp.float32),
    mesh=_MESH,
    scratch_types=[
        pltpu.VMEM((2, SEG_SUPER), jnp.int32),
        pltpu.VMEM((2, SEG_SUPER), jnp.float32),
        pltpu.VMEM((N_ACC,), jnp.float32),
        pltpu.SemaphoreType.DMA,
        pltpu.SemaphoreType.DMA,
    ],
    compiler_params=pltpu.CompilerParams(
        use_tc_tiling_on_sc=False, needs_layout_passes=False),
)
def _segsum_sc(ewt_hbm, ei_hbm, out_hbm, idx2, val2, acc, sem_a, sem_b):
    c = lax.axis_index("c")
    s = lax.axis_index("s")
    base = c * E_HALF

    zvec = jnp.zeros((16,), jnp.float32)

    def _zero(i, carry):
        acc[pl.ds(i * 16, 16)] = zvec
        return carry

    lax.fori_loop(0, N_ACC // 16, _zero, 0)

    sems = (sem_a, sem_b)

    def _fire(sup, b):
        off = base + sup * SEG_SUPER
        pltpu.async_copy(ei_hbm.at[0, pl.ds(off, SEG_SUPER)], idx2.at[b], sems[b])
        pltpu.async_copy(ewt_hbm.at[s, pl.ds(off, SEG_SUPER)], val2.at[b], sems[b])

    def _drain(b):
        pltpu.make_async_copy(ei_hbm.at[0, pl.ds(base, SEG_SUPER)], idx2.at[b], sems[b]).wait()
        pltpu.make_async_copy(ewt_hbm.at[s, pl.ds(base, SEG_SUPER)], val2.at[b], sems[b]).wait()

    _fire(0, 0)
    for sup in range(N_SEG_SUPER):
        b = sup % 2
        if sup + 1 < N_SEG_SUPER:
            _fire(sup + 1, 1 - b)
        _drain(b)
        idx_v = idx2.at[b]
        val_v = val2.at[b]

        @plsc.parallel_loop(0, SEG_SUPER // 16, unroll=8)
        def _group(i):
            sl = pl.ds(i * 16, 16)
            plsc.addupdate_scatter(acc, [idx_v[sl]], val_v[sl])

    pltpu.sync_copy(acc, out_hbm.at[c, s])


@functools.partial(
    pl.kernel,
    out_type=jax.ShapeDtypeStruct((N_EDGES, OUT_DIM), jnp.float32),
    mesh=_MESH,
    scratch_types=[
        pltpu.VMEM((E_PER,), jnp.int32),
        pltpu.VMEM((E_PER,), jnp.int32),
        pltpu.VMEM((3, CHUNK, OUT_DIM), jnp.bfloat16),
        pltpu.VMEM((3, CHUNK, OUT_DIM), jnp.bfloat16),
        pltpu.VMEM((3, CHUNK, OUT_DIM), jnp.bfloat16),
        pltpu.VMEM((3, CHUNK, OUT_DIM), jnp.float32),
        pltpu.SemaphoreType.DMA,
        pltpu.SemaphoreType.DMA,
        pltpu.SemaphoreType.DMA,
        pltpu.SemaphoreType.DMA,
        pltpu.SemaphoreType.DMA,
        pltpu.SemaphoreType.DMA,
    ],
    compiler_params=pltpu.CompilerParams(
        use_tc_tiling_on_sc=False, needs_layout_passes=False),
)
def _edge_sc(g_hbm, ei_hbm, mew_hbm, out_hbm,
             idx_s, idx_d, gs3, gd3, mw3, oo3,
             sem_0, sem_1, sem_2, sem_s0, sem_s1, sem_s2):
    c = lax.axis_index("c")
    s = lax.axis_index("s")
    wid = c * NS + s
    base = wid * E_PER

    pltpu.sync_copy(ei_hbm.at[0, pl.ds(base, E_PER)], idx_s)
    pltpu.sync_copy(ei_hbm.at[1, pl.ds(base, E_PER)], idx_d)

    bufs = ((gs3.at[0], gd3.at[0], mw3.at[0], oo3.at[0], sem_0, sem_s0),
            (gs3.at[1], gd3.at[1], mw3.at[1], oo3.at[1], sem_1, sem_s1),
            (gs3.at[2], gd3.at[2], mw3.at[2], oo3.at[2], sem_2, sem_s2))

    def _fire(k, b):
        gs, gd, mw, _, sem, _ = bufs[b]
        pltpu.async_copy(g_hbm.at[idx_s.at[pl.ds(k * CHUNK, CHUNK)]], gs, sem)
        pltpu.async_copy(g_hbm.at[idx_d.at[pl.ds(k * CHUNK, CHUNK)]], gd, sem)
        pltpu.async_copy(mew_hbm.at[pl.ds(base + k * CHUNK, CHUNK)], mw, sem)

    def _drain_loads(b):
        gs, gd, mw, _, sem, _ = bufs[b]
        pltpu.make_async_copy(g_hbm.at[idx_s.at[pl.ds(0, CHUNK)]], gs, sem).wait()
        pltpu.make_async_copy(g_hbm.at[idx_d.at[pl.ds(0, CHUNK)]], gd, sem).wait()
        pltpu.make_async_copy(mew_hbm.at[pl.ds(base, CHUNK)], mw, sem).wait()

    def _drain_store(b):
        _, _, _, oo, _, sems = bufs[b]
        pltpu.make_async_copy(oo, out_hbm.at[pl.ds(base, CHUNK)], sems).wait()

    def _compute(b):
        gs, gd, mw, oo, _, _ = bufs[b]

        @plsc.parallel_loop(0, CHUNK * (OUT_DIM // 32), unroll=4)
        def _group(i):
            r = i // (OUT_DIM // 32)
            u = i % (OUT_DIM // 32)
            sl32 = pl.ds(u * 32, 32)
            gsw = plsc.bitcast(gs[r, sl32], jnp.int32)
            gdw = plsc.bitcast(gd[r, sl32], jnp.int32)
            mww = plsc.bitcast(mw[r, sl32], jnp.int32)
            lo = (plsc.bitcast(gsw << 16, jnp.float32)
                  + plsc.bitcast(gdw << 16, jnp.float32)
                  + plsc.bitcast(mww << 16, jnp.float32))
            hi = (plsc.bitcast(gsw & -65536, jnp.float32)
                  + plsc.bitcast(gdw & -65536, jnp.float32)
                  + plsc.bitcast(mww & -65536, jnp.float32))
            oo[r, pl.ds(u * 32, 16)] = lo
            oo[r, pl.ds(u * 32 + 16, 16)] = hi

    def _fire_store(k, b):
        _, _, _, oo, _, sems = bufs[b]
        pltpu.async_copy(oo, out_hbm.at[pl.ds(base + k * CHUNK, CHUNK)], sems)

    _fire(0, 0)
    _fire(1, 1)

    def _triple(t, carry):
        k0 = 3 * t
        for j in range(3):
            k = k0 + j
            b = j
            _drain_loads(b)
            _compute(b)
            _fire_store(k, b)

            @pl.when(k >= 1)
            def _():
                _drain_store((j + 2) % 3)

            _fire(k + 2, (j + 2) % 3)
        return carry

    lax.fori_loop(0, (N_CHUNKS - 2) // 3, _triple, 0)

    _drain_loads(0)
    _compute(0)
    _fire_store(N_CHUNKS - 2, 0)
    _drain_store(2)
    _drain_loads(1)
    _compute(1)
    _fire_store(N_CHUNKS - 1, 1)
    _drain_store(0)
    _drain_store(1)


_DN_LHS_T = (((0,), (0,)), ((), ()))


def _mew_body(ewt_ref, w_ref, o_ref):
    o_ref[...] = lax.dot_general(
        ewt_ref[...], w_ref[...], _DN_LHS_T,
        preferred_element_type=jnp.float32).astype(jnp.bfloat16)


def _mew_tc(ewt, w_ew_i):
    blk = 16000
    return pl.pallas_call(
        _mew_body,
        grid=(N_EDGES // blk,),
        in_specs=[
            pl.BlockSpec((EW_DIM, blk), lambda i: (0, i)),
            pl.BlockSpec((EW_DIM, OUT_DIM), lambda i: (0, 0)),
        ],
        out_specs=pl.BlockSpec((blk, OUT_DIM), lambda i: (i, 0)),
        out_shape=jax.ShapeDtypeStruct((N_EDGES, OUT_DIM), jnp.bfloat16),
    )(ewt, w_ew_i)


def _g_body(x_ref, p_ref, wx_ref, wj_ref, o_ref):
    ssum = p_ref[0] + p_ref[1]
    o_ref[...] = (
        jnp.dot(x_ref[...], wx_ref[...], preferred_element_type=jnp.float32)
        + lax.dot_general(ssum, wj_ref[...], _DN_LHS_T,
                          preferred_element_type=jnp.float32)
    ).astype(jnp.bfloat16)


def _g_tc(x_pad, partials, w_x, w_ew_j):
    blk = 2560
    return pl.pallas_call(
        _g_body,
        grid=(N_ACC // blk,),
        in_specs=[
            pl.BlockSpec((blk, X_DIM), lambda i: (i, 0)),
            pl.BlockSpec((NC, EW_DIM, blk), lambda i: (0, 0, i)),
            pl.BlockSpec((X_DIM, OUT_DIM), lambda i: (0, 0)),
            pl.BlockSpec((EW_DIM, OUT_DIM), lambda i: (0, 0)),
        ],
        out_specs=pl.BlockSpec((blk, OUT_DIM), lambda i: (i, 0)),
        out_shape=jax.ShapeDtypeStruct((N_ACC, OUT_DIM), jnp.bfloat16),
    )(x_pad, partials, w_x, w_ew_j)


_Q = np.arange(128).reshape(4, 2, 16).transpose(0, 2, 1).reshape(128)


def kernel(x, edge_index, edge_weight, w_x, w_ew_i, w_ew_j):
    ei = edge_index.astype(jnp.int32)
    ewt = edge_weight.T
    q = jnp.asarray(_Q)

    partials = _segsum_sc(ewt, ei)
    mew = _mew_tc(ewt, w_ew_i[:, q])
    x_pad = jnp.pad(x, ((0, N_ACC - N_NODES), (0, 0)))
    g = _g_tc(x_pad, partials, w_x[:, q], w_ew_j[:, q])
    return _edge_sc(g, ei, mew)


# speedup vs baseline: 1.7079x; 1.7079x over previous
import functools

import jax
import jax.numpy as jnp
import numpy as np
from jax import lax
from jax.experimental import pallas as pl
from jax.experimental.pallas import tpu as pltpu
from jax.experimental.pallas import tpu_sc as plsc

N_NODES = 10000
N_EDGES = 320000
X_DIM = 128
EW_DIM = 16
OUT_DIM = 128

NC = 2
NS = 16
NW = NC * NS
E_PER = N_EDGES // NW
CHUNK = 80
N_CHUNKS = E_PER // CHUNK
N_ACC = 10240

E_HALF = N_EDGES // NC
SEG_SUPER = 20000
N_SEG_SUPER = E_HALF // SEG_SUPER

_MESH = plsc.VectorSubcoreMesh(
    core_axis_name="c", subcore_axis_name="s", num_cores=NC, num_subcores=NS)


@functools.partial(
    pl.kernel,
    out_type=jax.ShapeDtypeStruct((NC, EW_DIM, N_ACC), jnp.float32),
    mesh=_MESH,
    scratch_types=[
        pltpu.VMEM((2, SEG_SUPER), jnp.int32),
        pltpu.VMEM((2, SEG_SUPER), jnp.float32),
        pltpu.VMEM((N_ACC,), jnp.float32),
        pltpu.SemaphoreType.DMA,
        pltpu.SemaphoreType.DMA,
    ],
    compiler_params=pltpu.CompilerParams(
        use_tc_tiling_on_sc=False, needs_layout_passes=False),
)
def _segsum_sc(ewt_hbm, ei_hbm, out_hbm, idx2, val2, acc, sem_a, sem_b):
    c = lax.axis_index("c")
    s = lax.axis_index("s")
    base = c * E_HALF

    zvec = jnp.zeros((16,), jnp.float32)

    def _zero(i, carry):
        acc[pl.ds(i * 16, 16)] = zvec
        return carry

    lax.fori_loop(0, N_ACC // 16, _zero, 0)

    sems = (sem_a, sem_b)

    def _fire(sup, b):
        off = base + sup * SEG_SUPER
        pltpu.async_copy(ei_hbm.at[0, pl.ds(off, SEG_SUPER)], idx2.at[b], sems[b])
        pltpu.async_copy(ewt_hbm.at[s, pl.ds(off, SEG_SUPER)], val2.at[b], sems[b])

    def _drain(b):
        pltpu.make_async_copy(ei_hbm.at[0, pl.ds(base, SEG_SUPER)], idx2.at[b], sems[b]).wait()
        pltpu.make_async_copy(ewt_hbm.at[s, pl.ds(base, SEG_SUPER)], val2.at[b], sems[b]).wait()

    _fire(0, 0)
    for sup in range(N_SEG_SUPER):
        b = sup % 2
        if sup + 1 < N_SEG_SUPER:
            _fire(sup + 1, 1 - b)
        _drain(b)
        idx_v = idx2.at[b]
        val_v = val2.at[b]

        @plsc.parallel_loop(0, SEG_SUPER // 16, unroll=8)
        def _group(i):
            sl = pl.ds(i * 16, 16)
            plsc.addupdate_scatter(acc, [idx_v[sl]], val_v[sl])

    pltpu.sync_copy(acc, out_hbm.at[c, s])


@functools.partial(
    pl.kernel,
    out_type=jax.ShapeDtypeStruct((N_EDGES, OUT_DIM), jnp.float32),
    mesh=_MESH,
    scratch_types=[
        pltpu.VMEM((E_PER,), jnp.int32),
        pltpu.VMEM((E_PER,), jnp.int32),
        pltpu.VMEM((3, CHUNK, OUT_DIM // 2), jnp.int32),
        pltpu.VMEM((3, CHUNK, OUT_DIM // 2), jnp.int32),
        pltpu.VMEM((3, CHUNK, OUT_DIM), jnp.float32),
        pltpu.VMEM((3, CHUNK, OUT_DIM), jnp.float32),
        pltpu.SemaphoreType.DMA,
        pltpu.SemaphoreType.DMA,
        pltpu.SemaphoreType.DMA,
        pltpu.SemaphoreType.DMA,
        pltpu.SemaphoreType.DMA,
        pltpu.SemaphoreType.DMA,
    ],
    compiler_params=pltpu.CompilerParams(
        use_tc_tiling_on_sc=False, needs_layout_passes=False),
)
def _edge_sc(g_hbm, ei_hbm, mew_hbm, out_hbm,
             idx_s, idx_d, gs3, gd3, mw3, oo3,
             sem_0, sem_1, sem_2, sem_s0, sem_s1, sem_s2):
    c = lax.axis_index("c")
    s = lax.axis_index("s")
    wid = c * NS + s
    base = wid * E_PER

    pltpu.sync_copy(ei_hbm.at[0, pl.ds(base, E_PER)], idx_s)
    pltpu.sync_copy(ei_hbm.at[1, pl.ds(base, E_PER)], idx_d)

    bufs = ((gs3.at[0], gd3.at[0], mw3.at[0], oo3.at[0], sem_0, sem_s0),
            (gs3.at[1], gd3.at[1], mw3.at[1], oo3.at[1], sem_1, sem_s1),
            (gs3.at[2], gd3.at[2], mw3.at[2], oo3.at[2], sem_2, sem_s2))

    def _fire(k, b):
        gs, gd, mw, _, sem, _ = bufs[b]
        pltpu.async_copy(g_hbm.at[idx_s.at[pl.ds(k * CHUNK, CHUNK)]], gs, sem)
        pltpu.async_copy(g_hbm.at[idx_d.at[pl.ds(k * CHUNK, CHUNK)]], gd, sem)
        pltpu.async_copy(mew_hbm.at[pl.ds(base + k * CHUNK, CHUNK)], mw, sem)

    def _drain_loads(b):
        gs, gd, mw, _, sem, _ = bufs[b]
        pltpu.make_async_copy(g_hbm.at[idx_s.at[pl.ds(0, CHUNK)]], gs, sem).wait()
        pltpu.make_async_copy(g_hbm.at[idx_d.at[pl.ds(0, CHUNK)]], gd, sem).wait()
        pltpu.make_async_copy(mew_hbm.at[pl.ds(base, CHUNK)], mw, sem).wait()

    def _drain_store(b):
        _, _, _, oo, _, sems = bufs[b]
        pltpu.make_async_copy(oo, out_hbm.at[pl.ds(base, CHUNK)], sems).wait()

    def _compute(b):
        gs, gd, mw, oo, _, _ = bufs[b]

        @plsc.parallel_loop(0, CHUNK * (OUT_DIM // 32), unroll=4)
        def _group(i):
            r = i // (OUT_DIM // 32)
            u = i % (OUT_DIM // 32)
            gsw = gs[r, pl.ds(u * 16, 16)]
            gdw = gd[r, pl.ds(u * 16, 16)]
            lo = (plsc.bitcast(gsw << 16, jnp.float32)
                  + plsc.bitcast(gdw << 16, jnp.float32)
                  + mw[r, pl.ds(u * 32, 16)])
            hi = (plsc.bitcast(gsw & -65536, jnp.float32)
                  + plsc.bitcast(gdw & -65536, jnp.float32)
                  + mw[r, pl.ds(u * 32 + 16, 16)])
            oo[r, pl.ds(u * 32, 16)] = lo
            oo[r, pl.ds(u * 32 + 16, 16)] = hi

    def _fire_store(k, b):
        _, _, _, oo, _, sems = bufs[b]
        pltpu.async_copy(oo, out_hbm.at[pl.ds(base + k * CHUNK, CHUNK)], sems)

    _fire(0, 0)
    _fire(1, 1)

    def _triple(t, carry):
        k0 = 3 * t
        for j in range(3):
            k = k0 + j
            b = j
            _drain_loads(b)
            _compute(b)
            _fire_store(k, b)

            @pl.when(k >= 1)
            def _():
                _drain_store((j + 2) % 3)

            _fire(k + 2, (j + 2) % 3)
        return carry

    lax.fori_loop(0, (N_CHUNKS - 2) // 3, _triple, 0)

    _drain_loads(0)
    _compute(0)
    _fire_store(N_CHUNKS - 2, 0)
    _drain_store(2)
    _drain_loads(1)
    _compute(1)
    _fire_store(N_CHUNKS - 1, 1)
    _drain_store(0)
    _drain_store(1)


_DN_LHS_T = (((0,), (0,)), ((), ()))


def _mew_body(ewt_ref, w_ref, o_ref):
    o_ref[...] = lax.dot_general(ewt_ref[...], w_ref[...], _DN_LHS_T,
                                 preferred_element_type=jnp.float32)


def _mew_tc(ewt, w_ew_i):
    blk = 16000
    return pl.pallas_call(
        _mew_body,
        grid=(N_EDGES // blk,),
        in_specs=[
            pl.BlockSpec((EW_DIM, blk), lambda i: (0, i)),
            pl.BlockSpec((EW_DIM, OUT_DIM), lambda i: (0, 0)),
        ],
        out_specs=pl.BlockSpec((blk, OUT_DIM), lambda i: (i, 0)),
        out_shape=jax.ShapeDtypeStruct((N_EDGES, OUT_DIM), jnp.float32),
    )(ewt, w_ew_i)


def _g_body(x_ref, p_ref, wxa_ref, wxb_ref, wja_ref, wjb_ref, o_ref):
    ssum = p_ref[0] + p_ref[1]
    x = x_ref[...]
    av = (jnp.dot(x, wxa_ref[...], preferred_element_type=jnp.float32)
          + lax.dot_general(ssum, wja_ref[...], _DN_LHS_T,
                            preferred_element_type=jnp.float32))
    bv = (jnp.dot(x, wxb_ref[...], preferred_element_type=jnp.float32)
          + lax.dot_general(ssum, wjb_ref[...], _DN_LHS_T,
                            preferred_element_type=jnp.float32))
    ai = lax.bitcast_convert_type(av, jnp.int32) + 0x8000
    bi = lax.bitcast_convert_type(bv, jnp.int32) + 0x8000
    o_ref[...] = ((ai >> 16) & 0xFFFF) | (bi & -65536)


def _g_tc(x_pad, partials, wxa, wxb, wja, wjb):
    blk = 2560
    half = OUT_DIM // 2
    return pl.pallas_call(
        _g_body,
        grid=(N_ACC // blk,),
        in_specs=[
            pl.BlockSpec((blk, X_DIM), lambda i: (i, 0)),
            pl.BlockSpec((NC, EW_DIM, blk), lambda i: (0, 0, i)),
            pl.BlockSpec((X_DIM, half), lambda i: (0, 0)),
            pl.BlockSpec((X_DIM, half), lambda i: (0, 0)),
            pl.BlockSpec((EW_DIM, half), lambda i: (0, 0)),
            pl.BlockSpec((EW_DIM, half), lambda i: (0, 0)),
        ],
        out_specs=pl.BlockSpec((blk, half), lambda i: (i, 0)),
        out_shape=jax.ShapeDtypeStruct((N_ACC, half), jnp.int32),
    )(x_pad, partials, wxa, wxb, wja, wjb)


_U = np.arange(OUT_DIM // 2) // 16
_I = np.arange(OUT_DIM // 2) % 16
_COLS_A = 32 * _U + _I
_COLS_B = _COLS_A + 16


def kernel(x, edge_index, edge_weight, w_x, w_ew_i, w_ew_j):
    ei = edge_index.astype(jnp.int32)
    ewt = edge_weight.T
    ca = jnp.asarray(_COLS_A)
    cb = jnp.asarray(_COLS_B)

    partials = _segsum_sc(ewt, ei)
    mew = _mew_tc(ewt, w_ew_i)
    x_pad = jnp.pad(x, ((0, N_ACC - N_NODES), (0, 0)))
    g = _g_tc(x_pad, partials, w_x[:, ca], w_x[:, cb], w_ew_j[:, ca],
              w_ew_j[:, cb])
    return _edge_sc(g, ei, mew)
